# baseline (speedup 1.0000x reference)
import functools

import jax
import jax.numpy as jnp
from jax import lax
from jax.experimental import pallas as pl
from jax.experimental.pallas import tpu as pltpu
from jax.experimental.pallas import tpu_sc as plsc

B, S = 1024, 200
VOCAB, C, D = 100000, 4, 64
CD = C * D
CDW = CD // 2
WIN = 104
HS = S // 2
NC, NS = 2, 16
NW = NC * NS
TASKS_PER_W = B // NW


_HI = -65536


def _lo(w):
    return lax.bitcast_convert_type(lax.shift_left(w, 16), jnp.float32)


def _hi(w):
    return lax.bitcast_convert_type(lax.bitwise_and(w, _HI), jnp.float32)


def _row_sum(emb, p, g):
    return (_lo(emb[p, pl.ds(g * 16, 16)])
            + _lo(emb[p + 1, pl.ds(64 + g * 16, 16)])
            + _hi(emb[p + 2, pl.ds(g * 16, 16)])
            + _hi(emb[p + 3, pl.ds(64 + g * 16, 16)]))


def _compute(embA, embB, outb):
    def s_lo(r, _):
        for h in range(2):
            for g in range(D // 16):
                outb[r, pl.ds(h * D + g * 16, 16)] = _row_sum(
                    embA, 2 * r + h, g)
        return 0

    lax.fori_loop(0, HS // 2, s_lo, 0, unroll=2)

    def s_hi(r, _):
        for h in range(2):
            for g in range(D // 16):
                outb[r, pl.ds(h * D + g * 16, 16)] = _row_sum(
                    embB, 2 * r + h - HS, g)
        return 0

    lax.fori_loop(HS // 2, S // 2, s_hi, 0, unroll=2)


def _sc_body(x2_hbm, table_hbm, out_hbm, idx0, idx1,
             embA0, embB0, embA1, embB1, outb,
             si0, si1, sg0, sg1, so):
    wid = lax.axis_index("s") * NC + lax.axis_index("c")
    base = wid * TASKS_PER_W

    def idx_fetch(t, idxb, sem):
        return pltpu.async_copy(x2_hbm.at[base + t], idxb, sem)

    def idx_wait(t, idxb, sem):
        pltpu.make_async_copy(x2_hbm.at[base + t], idxb, sem).wait()

    def gather(idxb, embAb, embBb, sem):
        pltpu.async_copy(table_hbm.at[idxb.at[0]], embAb, sem)
        pltpu.async_copy(table_hbm.at[idxb.at[1]], embBb, sem)

    def gather_wait(idxb, embAb, embBb, sem):
        pltpu.make_async_copy(table_hbm.at[idxb.at[0]], embAb, sem).wait()
        pltpu.make_async_copy(table_hbm.at[idxb.at[1]], embBb, sem).wait()

    def scatter(t, sem):
        return pltpu.async_copy(outb, out_hbm.at[base + t], sem)

    def scatter_wait(t, sem):
        pltpu.make_async_copy(outb, out_hbm.at[base + t], sem).wait()

    pltpu.sync_copy(x2_hbm.at[base], idx0)
    gather(idx0, embA0, embB0, sg0)
    idx_fetch(1, idx1, si1)

    def task_body(k, _):
        t0 = 2 * k
        idx_wait(t0 + 1, idx1, si1)
        gather(idx1, embA1, embB1, sg1)
        gather_wait(idx0, embA0, embB0, sg0)

        @pl.when(k < TASKS_PER_W // 2 - 1)
        def _():
            idx_fetch(t0 + 2, idx0, si0)

        @pl.when(k >= 1)
        def _():
            scatter_wait(t0 - 1, so)

        _compute(embA0, embB0, outb)
        scatter(t0, so)

        @pl.when(k < TASKS_PER_W // 2 - 1)
        def _():
            idx_wait(t0 + 2, idx0, si0)
            gather(idx0, embA0, embB0, sg0)

        gather_wait(idx1, embA1, embB1, sg1)

        @pl.when(k < TASKS_PER_W // 2 - 1)
        def _():
            idx_fetch(t0 + 3, idx1, si1)

        scatter_wait(t0, so)
        _compute(embA1, embB1, outb)
        scatter(t0 + 1, so)
        return 0

    lax.fori_loop(0, TASKS_PER_W // 2, task_body, 0)
    scatter_wait(TASKS_PER_W - 1, so)


def kernel(x, table):
    x = x.astype(jnp.int32)
    x2 = jnp.concatenate([x, jnp.zeros((B, C), jnp.int32)], axis=1)
    x2win = jnp.stack([x2[:, :WIN], x2[:, S - HS:]], axis=1)
    bits = lax.bitcast_convert_type(table.reshape(VOCAB, CD), jnp.int32)
    lo = lax.shift_right_logical(bits[:, :CDW] + 0x8000, 16)
    hi = lax.bitwise_and(bits[:, CDW:] + 0x8000, _HI)
    table2d = lax.bitwise_or(hi, lo)

    mesh = plsc.VectorSubcoreMesh(core_axis_name="c", subcore_axis_name="s")
    run = functools.partial(
        pl.kernel,
        mesh=mesh,
        out_type=jax.ShapeDtypeStruct((B, S // 2, 2 * D), jnp.float32),
        scratch_types=[
            pltpu.VMEM((2, WIN), jnp.int32),
            pltpu.VMEM((2, WIN), jnp.int32),
            pltpu.VMEM((WIN, CDW), jnp.int32),
            pltpu.VMEM((WIN, CDW), jnp.int32),
            pltpu.VMEM((WIN, CDW), jnp.int32),
            pltpu.VMEM((WIN, CDW), jnp.int32),
            pltpu.VMEM((S // 2, 2 * D), jnp.float32),
            pltpu.SemaphoreType.DMA,
            pltpu.SemaphoreType.DMA,
            pltpu.SemaphoreType.DMA,
            pltpu.SemaphoreType.DMA,
            pltpu.SemaphoreType.DMA,
        ],
    )(_sc_body)
    return run(x2win, table2d).reshape(B, S, D)


# speedup vs baseline: 1.2696x; 1.2696x over previous
import functools

import jax
import jax.numpy as jnp
from jax import lax
from jax.experimental import pallas as pl
from jax.experimental.pallas import tpu as pltpu
from jax.experimental.pallas import tpu_sc as plsc

B, S = 1024, 200
VOCAB, C, D = 100000, 4, 64
CD = C * D
WIN = 104
HS = S // 2
NC, NS = 2, 16
NW = NC * NS
TASKS_PER_W = B // NW


def _acc_row(loads):
    acc = loads[0]
    for v in loads[1:]:
        acc = acc + v
    return acc


def _compute(embA, embB, outb):
    def s_lo(r, _):
        for h in range(2):
            for g in range(D // 16):
                outb[r, pl.ds(h * D + g * 16, 16)] = _acc_row(
                    [embA[2 * r + h + i, pl.ds(i * D + g * 16, 16)]
                     for i in range(C)])
        return 0

    lax.fori_loop(0, HS // 2, s_lo, 0, unroll=2)

    def s_hi(r, _):
        for h in range(2):
            for g in range(D // 16):
                outb[r, pl.ds(h * D + g * 16, 16)] = _acc_row(
                    [embB[2 * r + h + i - HS, pl.ds(i * D + g * 16, 16)]
                     for i in range(C)])
        return 0

    lax.fori_loop(HS // 2, S // 2, s_hi, 0, unroll=2)


def _sc_body(x2_hbm, table_hbm, out_hbm, idx0, idx1,
             embA0, embB0, embA1, embB1, outb,
             si0, si1, sg0, sg1, so):
    wid = lax.axis_index("s") * NC + lax.axis_index("c")
    base = wid * TASKS_PER_W

    def idx_fetch(t, idxb, sem):
        return pltpu.async_copy(x2_hbm.at[base + t], idxb, sem)

    def idx_wait(t, idxb, sem):
        pltpu.make_async_copy(x2_hbm.at[base + t], idxb, sem).wait()

    def gather(idxb, embAb, embBb, sem):
        pltpu.async_copy(table_hbm.at[idxb.at[0]], embAb, sem)
        pltpu.async_copy(table_hbm.at[idxb.at[1]], embBb, sem)

    def gather_wait(idxb, embAb, embBb, sem):
        pltpu.make_async_copy(table_hbm.at[idxb.at[0]], embAb, sem).wait()
        pltpu.make_async_copy(table_hbm.at[idxb.at[1]], embBb, sem).wait()

    def scatter(t, sem):
        return pltpu.async_copy(outb, out_hbm.at[base + t], sem)

    def scatter_wait(t, sem):
        pltpu.make_async_copy(outb, out_hbm.at[base + t], sem).wait()

    pltpu.sync_copy(x2_hbm.at[base], idx0)
    gather(idx0, embA0, embB0, sg0)
    idx_fetch(1, idx1, si1)

    def task_body(k, _):
        t0 = 2 * k
        idx_wait(t0 + 1, idx1, si1)
        gather(idx1, embA1, embB1, sg1)
        gather_wait(idx0, embA0, embB0, sg0)

        @pl.when(k < TASKS_PER_W // 2 - 1)
        def _():
            idx_fetch(t0 + 2, idx0, si0)

        @pl.when(k >= 1)
        def _():
            scatter_wait(t0 - 1, so)

        _compute(embA0, embB0, outb)
        scatter(t0, so)

        @pl.when(k < TASKS_PER_W // 2 - 1)
        def _():
            idx_wait(t0 + 2, idx0, si0)
            gather(idx0, embA0, embB0, sg0)

        gather_wait(idx1, embA1, embB1, sg1)

        @pl.when(k < TASKS_PER_W // 2 - 1)
        def _():
            idx_fetch(t0 + 3, idx1, si1)

        scatter_wait(t0, so)
        _compute(embA1, embB1, outb)
        scatter(t0 + 1, so)
        return 0

    lax.fori_loop(0, TASKS_PER_W // 2, task_body, 0)
    scatter_wait(TASKS_PER_W - 1, so)


def kernel(x, table):
    x = x.astype(jnp.int32)
    x2 = jnp.concatenate([x, jnp.zeros((B, C), jnp.int32)], axis=1)
    x2win = jnp.stack([x2[:, :WIN], x2[:, S - HS:]], axis=1)
    table2d = table.reshape(VOCAB, CD)

    mesh = plsc.VectorSubcoreMesh(core_axis_name="c", subcore_axis_name="s")
    run = functools.partial(
        pl.kernel,
        mesh=mesh,
        out_type=jax.ShapeDtypeStruct((B, S // 2, 2 * D), jnp.float32),
        scratch_types=[
            pltpu.VMEM((2, WIN), jnp.int32),
            pltpu.VMEM((2, WIN), jnp.int32),
            pltpu.VMEM((WIN, CD), jnp.float32),
            pltpu.VMEM((WIN, CD), jnp.float32),
            pltpu.VMEM((WIN, CD), jnp.float32),
            pltpu.VMEM((WIN, CD), jnp.float32),
            pltpu.VMEM((S // 2, 2 * D), jnp.float32),
            pltpu.SemaphoreType.DMA,
            pltpu.SemaphoreType.DMA,
            pltpu.SemaphoreType.DMA,
            pltpu.SemaphoreType.DMA,
            pltpu.SemaphoreType.DMA,
        ],
    )(_sc_body)
    return run(x2win, table2d).reshape(B, S, D)
